# baseline (speedup 1.0000x reference)
import functools

import jax
import jax.numpy as jnp
from jax import lax
from jax.experimental import pallas as pl
from jax.experimental.pallas import tpu as pltpu
from jax.experimental.pallas import tpu_sc as plsc

_TEMPERATURE = 2.2
_TINY = float(jnp.finfo(jnp.float32).tiny)
_ROT_A = (13, 15, 26, 6)
_ROT_B = (17, 29, 16, 24)
_KS = (0, 42, 0x1BD11BDA ^ 42)
_INJ = tuple(
    (_KS[(i + 1) % 3], (_KS[(i + 2) % 3] + i + 1) & 0xFFFFFFFF) for i in range(5)
)

_ROWS = 32
_VOCAB = 1_000_000
_TC_BLOCK = 32768
_TC_NBLK = 22
_HEAD = _TC_BLOCK * _TC_NBLK
_TAIL = _VOCAB - _HEAD
_SC_CHUNK = 8192
_SC_FULL_CHUNKS = _TAIL // _SC_CHUNK
_SC_REM = _TAIL - _SC_FULL_CHUNKS * _SC_CHUNK
_TAIL_BLOCK = 16384
_TAIL_NBLK = pl.cdiv(_TAIL, _TAIL_BLOCK)


def _threefry_bits(x1_init):
    x1 = x1_init
    x0 = x1
    x1 = ((x1 << jnp.uint32(13)) | (x1 >> jnp.uint32(19))) ^ x0
    for r in _ROT_A[1:]:
        x0 = x0 + x1
        x1 = ((x1 << jnp.uint32(r)) | (x1 >> jnp.uint32(32 - r))) ^ x0
    x0 = x0 + jnp.uint32(_INJ[0][0])
    x1 = x1 + jnp.uint32(_INJ[0][1])
    for i in range(1, 5):
        rots = _ROT_A if i % 2 == 0 else _ROT_B
        for r in rots:
            x0 = x0 + x1
            x1 = ((x1 << jnp.uint32(r)) | (x1 >> jnp.uint32(32 - r))) ^ x0
        x0 = x0 + jnp.uint32(_INJ[i][0])
        x1 = x1 + jnp.uint32(_INJ[i][1])
    return x0 ^ x1


def _gumbel_from_bits(bits):
    mant = (bits >> jnp.uint32(9)) | jnp.uint32(0x3F800000)
    f = jax.lax.bitcast_convert_type(mant, jnp.float32) - jnp.float32(1.0)
    u = jnp.maximum(jnp.float32(_TINY), f + jnp.float32(_TINY))
    return -jnp.log(-jnp.log(u))


def _sc_bits_body(out_hbm, buf):
    w = lax.axis_index("s") * 2 + lax.axis_index("c")
    lanes = lax.convert_element_type(lax.iota(jnp.int32, 16), jnp.uint32)
    row_base = w * _VOCAB + _HEAD + 42

    def fill(buf_len, chunk_off):
        def body(j, _):
            base = lax.convert_element_type(row_base + chunk_off + j * 16,
                                            jnp.uint32)
            bits = _threefry_bits(lanes + base)
            buf[pl.ds(j * 16, 16)] = bits
            return 0

        lax.fori_loop(0, buf_len // 16, body, 0)

    def chunk_body(ci, _):
        off = ci * _SC_CHUNK
        fill(_SC_CHUNK, off)
        pltpu.sync_copy(buf, out_hbm.at[pl.ds(w * _TAIL + off, _SC_CHUNK)])
        return 0

    lax.fori_loop(0, _SC_FULL_CHUNKS, chunk_body, 0)
    if _SC_REM:
        off = _SC_FULL_CHUNKS * _SC_CHUNK
        fill(_SC_REM, off)
        pltpu.sync_copy(
            buf.at[pl.ds(0, _SC_REM)],
            out_hbm.at[pl.ds(w * _TAIL + off, _SC_REM)],
        )


def _sc_tail_bits():
    mesh = plsc.VectorSubcoreMesh(core_axis_name="c", subcore_axis_name="s")
    return pl.kernel(
        _sc_bits_body,
        mesh=mesh,
        out_type=jax.ShapeDtypeStruct((_ROWS * _TAIL,), jnp.uint32),
        scratch_types=[pltpu.VMEM((_SC_CHUNK,), jnp.uint32)],
    )()


def _head_body(x_ref, ov_ref, oi_ref, rm_ref, ri_ref):
    pid = pl.program_id(0)
    nblk = pl.num_programs(0)
    blk = x_ref[...]
    rows, cw = blk.shape

    loc = jax.lax.broadcasted_iota(jnp.int32, (rows, cw), 1)
    col = loc + pid * _TC_BLOCK
    row = jax.lax.broadcasted_iota(jnp.int32, (rows, cw), 0)
    x1_init = (row * _VOCAB + col + 42).astype(jnp.uint32)
    g = _gumbel_from_bits(_threefry_bits(x1_init))

    pert = blk / jnp.float32(_TEMPERATURE) + g

    rm_old = jnp.where(pid == 0, -jnp.inf, rm_ref[...])
    upd = pert > rm_old
    rm_ref[...] = jnp.where(upd, pert, rm_old)
    ri_ref[...] = jnp.where(upd, col, ri_ref[...])

    @pl.when(pid == nblk - 1)
    def _():
        rm = rm_ref[...]
        m = jnp.max(rm, axis=1, keepdims=True)
        cand = jnp.where(rm == m, ri_ref[...], jnp.int32(0x7FFFFFFF))
        ov_ref[...] = m
        oi_ref[...] = jnp.min(cand, axis=1, keepdims=True)


def _head_scan(logits):
    return pl.pallas_call(
        _head_body,
        grid=(_TC_NBLK,),
        in_specs=[pl.BlockSpec((_ROWS, _TC_BLOCK), lambda i: (0, i))],
        out_specs=[
            pl.BlockSpec((_ROWS, 1), lambda i: (0, 0)),
            pl.BlockSpec((_ROWS, 1), lambda i: (0, 0)),
        ],
        out_shape=[
            jax.ShapeDtypeStruct((_ROWS, 1), jnp.float32),
            jax.ShapeDtypeStruct((_ROWS, 1), jnp.int32),
        ],
        scratch_shapes=[
            pltpu.VMEM((_ROWS, _TC_BLOCK), jnp.float32),
            pltpu.VMEM((_ROWS, _TC_BLOCK), jnp.int32),
        ],
        compiler_params=pltpu.CompilerParams(
            dimension_semantics=("arbitrary",),
        ),
    )(logits)


def _tail_body(x_ref, b_ref, hv_ref, hi_ref, o_ref, rm_ref, ri_ref):
    pid = pl.program_id(0)
    nblk = pl.num_programs(0)
    blk = x_ref[...]
    rows, cw = blk.shape

    loc = jax.lax.broadcasted_iota(jnp.int32, (rows, cw), 1)
    col = loc + (_HEAD + pid * _TAIL_BLOCK)
    g = _gumbel_from_bits(b_ref[...])

    pert = blk / jnp.float32(_TEMPERATURE) + g

    rm_old = jnp.where(pid == 0, -jnp.inf, rm_ref[...])
    upd = (pert > rm_old) & (loc < _TAIL - pid * _TAIL_BLOCK)
    rm_ref[...] = jnp.where(upd, pert, rm_old)
    ri_ref[...] = jnp.where(upd, col, ri_ref[...])

    @pl.when(pid == nblk - 1)
    def _():
        rm = rm_ref[...]
        m = jnp.max(rm, axis=1, keepdims=True)
        cand = jnp.where(rm == m, ri_ref[...], jnp.int32(0x7FFFFFFF))
        ti = jnp.min(cand, axis=1, keepdims=True)
        o_ref[...] = jnp.where(m > hv_ref[...], ti, hi_ref[...])


def _tail_scan(logits, bits, head_val, head_idx):
    return pl.pallas_call(
        _tail_body,
        grid=(_TAIL_NBLK,),
        in_specs=[
            pl.BlockSpec(
                (_ROWS, _TAIL_BLOCK), lambda i: (0, i + _HEAD // _TAIL_BLOCK)
            ),
            pl.BlockSpec((_ROWS, _TAIL_BLOCK), lambda i: (0, i)),
            pl.BlockSpec((_ROWS, 1), lambda i: (0, 0)),
            pl.BlockSpec((_ROWS, 1), lambda i: (0, 0)),
        ],
        out_specs=pl.BlockSpec((_ROWS, 1), lambda i: (0, 0)),
        out_shape=jax.ShapeDtypeStruct((_ROWS, 1), jnp.int32),
        scratch_shapes=[
            pltpu.VMEM((_ROWS, _TAIL_BLOCK), jnp.float32),
            pltpu.VMEM((_ROWS, _TAIL_BLOCK), jnp.int32),
        ],
        compiler_params=pltpu.CompilerParams(
            dimension_semantics=("arbitrary",),
        ),
    )(logits, bits, head_val, head_idx)


@jax.jit
def kernel(logits):
    rows, vocab = logits.shape
    tail_bits = _sc_tail_bits().reshape(_ROWS, _TAIL)
    head_val, head_idx = _head_scan(logits)
    out = _tail_scan(logits, tail_bits, head_val, head_idx)
    return out.reshape(rows)


# speedup vs baseline: 1.8758x; 1.8758x over previous
import functools

import jax
import jax.numpy as jnp
from jax import lax
from jax.experimental import pallas as pl
from jax.experimental.pallas import tpu as pltpu
from jax.experimental.pallas import tpu_sc as plsc

_TEMPERATURE = 2.2
_TINY = float(jnp.finfo(jnp.float32).tiny)
_ROT_A = (13, 15, 26, 6)
_ROT_B = (17, 29, 16, 24)
_KS = (0, 42, 0x1BD11BDA ^ 42)
_INJ = tuple(
    (_KS[(i + 1) % 3], (_KS[(i + 2) % 3] + i + 1) & 0xFFFFFFFF) for i in range(5)
)

_ROWS = 32
_VOCAB = 1_000_000
_TC_BLOCK = 32768
_TC_NBLK = 25
_HEAD = _TC_BLOCK * _TC_NBLK
_TAIL = _VOCAB - _HEAD
_SC_CHUNK = 8192
_SC_FULL_CHUNKS = _TAIL // _SC_CHUNK
_SC_REM = _TAIL - _SC_FULL_CHUNKS * _SC_CHUNK
_TAIL_BLOCK = 16384
_TAIL_NBLK = pl.cdiv(_TAIL, _TAIL_BLOCK)


def _threefry_bits(x1_init):
    x1 = x1_init
    x0 = x1
    x1 = ((x1 << jnp.uint32(13)) | (x1 >> jnp.uint32(19))) ^ x0
    for r in _ROT_A[1:]:
        x0 = x0 + x1
        x1 = ((x1 << jnp.uint32(r)) | (x1 >> jnp.uint32(32 - r))) ^ x0
    x0 = x0 + jnp.uint32(_INJ[0][0])
    x1 = x1 + jnp.uint32(_INJ[0][1])
    for i in range(1, 5):
        rots = _ROT_A if i % 2 == 0 else _ROT_B
        for r in rots:
            x0 = x0 + x1
            x1 = ((x1 << jnp.uint32(r)) | (x1 >> jnp.uint32(32 - r))) ^ x0
        x0 = x0 + jnp.uint32(_INJ[i][0])
        x1 = x1 + jnp.uint32(_INJ[i][1])
    return x0 ^ x1


def _gumbel_from_bits(bits):
    mant = (bits >> jnp.uint32(9)) | jnp.uint32(0x3F800000)
    f = jax.lax.bitcast_convert_type(mant, jnp.float32) - jnp.float32(1.0)
    u = jnp.maximum(jnp.float32(_TINY), f + jnp.float32(_TINY))
    return -jnp.log(-jnp.log(u))


def _sc_bits_body(out_hbm, buf):
    w = lax.axis_index("s") * 2 + lax.axis_index("c")
    lanes = lax.convert_element_type(lax.iota(jnp.int32, 16), jnp.uint32)
    row_base = w * _VOCAB + _HEAD + 42

    def fill(buf_len, chunk_off):
        def body(j, _):
            base = lax.convert_element_type(row_base + chunk_off + j * 16,
                                            jnp.uint32)
            bits = _threefry_bits(lanes + base)
            buf[pl.ds(j * 16, 16)] = bits
            return 0

        lax.fori_loop(0, buf_len // 16, body, 0)

    def chunk_body(ci, _):
        off = ci * _SC_CHUNK
        fill(_SC_CHUNK, off)
        pltpu.sync_copy(buf, out_hbm.at[pl.ds(w * _TAIL + off, _SC_CHUNK)])
        return 0

    lax.fori_loop(0, _SC_FULL_CHUNKS, chunk_body, 0)
    if _SC_REM:
        off = _SC_FULL_CHUNKS * _SC_CHUNK
        fill(_SC_REM, off)
        pltpu.sync_copy(
            buf.at[pl.ds(0, _SC_REM)],
            out_hbm.at[pl.ds(w * _TAIL + off, _SC_REM)],
        )


def _sc_tail_bits():
    mesh = plsc.VectorSubcoreMesh(core_axis_name="c", subcore_axis_name="s")
    return pl.kernel(
        _sc_bits_body,
        mesh=mesh,
        out_type=jax.ShapeDtypeStruct((_ROWS * _TAIL,), jnp.uint32),
        scratch_types=[pltpu.VMEM((_SC_CHUNK,), jnp.uint32)],
    )()


def _head_body(x_ref, ov_ref, oi_ref, rm_ref, ri_ref):
    pid = pl.program_id(0)
    nblk = pl.num_programs(0)
    blk = x_ref[...]
    rows, cw = blk.shape

    loc = jax.lax.broadcasted_iota(jnp.int32, (rows, cw), 1)
    col = loc + pid * _TC_BLOCK
    row = jax.lax.broadcasted_iota(jnp.int32, (rows, cw), 0)
    x1_init = (row * _VOCAB + col + 42).astype(jnp.uint32)
    g = _gumbel_from_bits(_threefry_bits(x1_init))

    pert = blk / jnp.float32(_TEMPERATURE) + g

    rm_old = jnp.where(pid == 0, -jnp.inf, rm_ref[...])
    upd = pert > rm_old
    rm_ref[...] = jnp.where(upd, pert, rm_old)
    ri_ref[...] = jnp.where(upd, col, ri_ref[...])

    @pl.when(pid == nblk - 1)
    def _():
        rm = rm_ref[...]
        m = jnp.max(rm, axis=1, keepdims=True)
        cand = jnp.where(rm == m, ri_ref[...], jnp.int32(0x7FFFFFFF))
        ov_ref[...] = m
        oi_ref[...] = jnp.min(cand, axis=1, keepdims=True)


def _head_scan(logits):
    return pl.pallas_call(
        _head_body,
        grid=(_TC_NBLK,),
        in_specs=[pl.BlockSpec((_ROWS, _TC_BLOCK), lambda i: (0, i))],
        out_specs=[
            pl.BlockSpec((_ROWS, 1), lambda i: (0, 0)),
            pl.BlockSpec((_ROWS, 1), lambda i: (0, 0)),
        ],
        out_shape=[
            jax.ShapeDtypeStruct((_ROWS, 1), jnp.float32),
            jax.ShapeDtypeStruct((_ROWS, 1), jnp.int32),
        ],
        scratch_shapes=[
            pltpu.VMEM((_ROWS, _TC_BLOCK), jnp.float32),
            pltpu.VMEM((_ROWS, _TC_BLOCK), jnp.int32),
        ],
        compiler_params=pltpu.CompilerParams(
            dimension_semantics=("arbitrary",),
        ),
    )(logits)


def _tail_body(x_ref, b_ref, hv_ref, hi_ref, o_ref, rm_ref, ri_ref):
    pid = pl.program_id(0)
    nblk = pl.num_programs(0)
    blk = x_ref[...]
    rows, cw = blk.shape

    loc = jax.lax.broadcasted_iota(jnp.int32, (rows, cw), 1)
    col = loc + (_HEAD + pid * _TAIL_BLOCK)
    g = _gumbel_from_bits(b_ref[...])

    pert = blk / jnp.float32(_TEMPERATURE) + g

    rm_old = jnp.where(pid == 0, -jnp.inf, rm_ref[...])
    upd = (pert > rm_old) & (loc < _TAIL - pid * _TAIL_BLOCK)
    rm_ref[...] = jnp.where(upd, pert, rm_old)
    ri_ref[...] = jnp.where(upd, col, ri_ref[...])

    @pl.when(pid == nblk - 1)
    def _():
        rm = rm_ref[...]
        m = jnp.max(rm, axis=1, keepdims=True)
        cand = jnp.where(rm == m, ri_ref[...], jnp.int32(0x7FFFFFFF))
        ti = jnp.min(cand, axis=1, keepdims=True)
        o_ref[...] = jnp.where(m > hv_ref[...], ti, hi_ref[...])


def _tail_scan(logits, bits, head_val, head_idx):
    return pl.pallas_call(
        _tail_body,
        grid=(_TAIL_NBLK,),
        in_specs=[
            pl.BlockSpec(
                (_ROWS, _TAIL_BLOCK), lambda i: (0, i + _HEAD // _TAIL_BLOCK)
            ),
            pl.BlockSpec((_ROWS, _TAIL_BLOCK), lambda i: (0, i)),
            pl.BlockSpec((_ROWS, 1), lambda i: (0, 0)),
            pl.BlockSpec((_ROWS, 1), lambda i: (0, 0)),
        ],
        out_specs=pl.BlockSpec((_ROWS, 1), lambda i: (0, 0)),
        out_shape=jax.ShapeDtypeStruct((_ROWS, 1), jnp.int32),
        scratch_shapes=[
            pltpu.VMEM((_ROWS, _TAIL_BLOCK), jnp.float32),
            pltpu.VMEM((_ROWS, _TAIL_BLOCK), jnp.int32),
        ],
        compiler_params=pltpu.CompilerParams(
            dimension_semantics=("arbitrary",),
        ),
    )(logits, bits, head_val, head_idx)


@jax.jit
def kernel(logits):
    rows, vocab = logits.shape
    tail_bits = _sc_tail_bits().reshape(_ROWS, _TAIL)
    head_val, head_idx = _head_scan(logits)
    out = _tail_scan(logits, tail_bits, head_val, head_idx)
    return out.reshape(rows)
